# Initial kernel scaffold; baseline (speedup 1.0000x reference)
import jax
import jax.numpy as jnp
from jax.experimental import pallas as pl


def kernel(x, W, b):
    raise NotImplementedError("write your pallas kernel here")


# speedup vs baseline: 4.9123x; 4.9123x over previous
import functools

import jax
import jax.numpy as jnp
from jax.experimental import pallas as pl
from jax.experimental.pallas import tpu as pltpu

TOKENS = 16384
D_MODEL = 4096
N_EXPERTS = 64
TOP_K = 8
TB = 512


def _body(x_ref, w_ref, b_ref, gs_ref, idx_ref, s_ref):
    z = jnp.dot(x_ref[...], w_ref[...], preferred_element_type=jnp.float32)
    z = z + b_ref[...]
    s = jax.nn.sigmoid(z)
    s_ref[...] = s

    iota = jax.lax.broadcasted_iota(jnp.int32, (TB, N_EXPERTS), 1)
    cur = s
    vals = []
    idxs = []
    for _ in range(TOP_K):
        v = jnp.max(cur, axis=1, keepdims=True)
        i = jnp.min(jnp.where(cur == v, iota, N_EXPERTS), axis=1, keepdims=True)
        vals.append(v)
        idxs.append(i)
        cur = jnp.where(iota == i, -jnp.inf, cur)
    g = jnp.concatenate(vals, axis=1)
    gs_ref[...] = g / jnp.sum(g, axis=1, keepdims=True)
    idx_ref[...] = jnp.concatenate(idxs, axis=1)


@jax.jit
def kernel(x, W, b):
    grid = (TOKENS // TB,)
    gs, idx, s = pl.pallas_call(
        _body,
        grid=grid,
        in_specs=[
            pl.BlockSpec((TB, D_MODEL), lambda t: (t, 0)),
            pl.BlockSpec((D_MODEL, N_EXPERTS), lambda t: (0, 0)),
            pl.BlockSpec((1, N_EXPERTS), lambda t: (0, 0)),
        ],
        out_specs=[
            pl.BlockSpec((TB, TOP_K), lambda t: (t, 0)),
            pl.BlockSpec((TB, TOP_K), lambda t: (t, 0)),
            pl.BlockSpec((TB, N_EXPERTS), lambda t: (t, 0)),
        ],
        out_shape=[
            jax.ShapeDtypeStruct((TOKENS, TOP_K), jnp.float32),
            jax.ShapeDtypeStruct((TOKENS, TOP_K), jnp.int32),
            jax.ShapeDtypeStruct((TOKENS, N_EXPERTS), jnp.float32),
        ],
        compiler_params=pltpu.CompilerParams(
            dimension_semantics=("arbitrary",),
        ),
    )(x, W, b.reshape(1, N_EXPERTS))
    return (gs, idx, s)
